# Initial kernel scaffold; baseline (speedup 1.0000x reference)
import jax
import jax.numpy as jnp
from jax.experimental import pallas as pl


def kernel(item_seq, feat_category, feat_tags, cat_table, tag_table):
    raise NotImplementedError("write your pallas kernel here")


# speedup vs baseline: 3.1923x; 3.1923x over previous
import functools

import jax
import jax.numpy as jnp
from jax import lax
from jax.experimental import pallas as pl
from jax.experimental.pallas import tpu as pltpu
from jax.experimental.pallas import tpu_sc as plsc

B = 4096
L = 50
D = 64
T = B * L
CAT_V = 1000
TAG_V = 2000
TAG_LEN = 5
FEAT_W = 16

NC = 2
NS = 16
NW = NC * NS
DH = D // 2
TOK_W = T // (NW // 2)
CHUNK = 128
NCHUNK = TOK_W // CHUNK
GROUPS = CHUNK // 16

_f32 = jnp.float32
_i32 = jnp.int32


def _build_kernel():
  mesh = plsc.VectorSubcoreMesh(core_axis_name="c", subcore_axis_name="s")

  @functools.partial(
      pl.kernel,
      out_type=(jax.ShapeDtypeStruct((T, D), _f32),
                jax.ShapeDtypeStruct((T, D), _f32)),
      mesh=mesh,
      compiler_params=pltpu.CompilerParams(use_tc_tiling_on_sc=False,
                                           needs_layout_passes=False),
      scratch_types=[
          pltpu.VMEM((CAT_V, DH), _f32),
          pltpu.VMEM((TAG_V, DH), _f32),
          pltpu.VMEM((CHUNK,), _i32),
          pltpu.VMEM((CHUNK,), _i32),
          pltpu.VMEM((CHUNK, FEAT_W), _i32),
          pltpu.VMEM((CHUNK, FEAT_W), _i32),
          pltpu.VMEM((CHUNK, DH), _f32),
          pltpu.VMEM((CHUNK, DH), _f32),
          pltpu.VMEM((CHUNK, DH), _f32),
          pltpu.VMEM((CHUNK, DH), _f32),
          pltpu.SemaphoreType.DMA,
          pltpu.SemaphoreType.DMA,
          pltpu.SemaphoreType.DMA,
          pltpu.SemaphoreType.DMA,
          pltpu.SemaphoreType.DMA,
          pltpu.SemaphoreType.DMA,
      ],
  )
  def seq_emb(seq_hbm, feat_hbm, ctab_hbm, ttab_hbm,
              out_cat_hbm, out_tag_hbm,
              ctab_v, ttab_v,
              ids0, ids1, tids0, tids1,
              oc0, oc1, ot0, ot1,
              sem_t0, sem_t1,
              sem_oc0, sem_oc1, sem_ot0, sem_ot1):
    wid = lax.axis_index("s") * NC + lax.axis_index("c")
    half = wid % 2
    tok_base = (wid // 2) * TOK_W
    c0 = half * DH

    ids = (ids0, ids1)
    tids = (tids0, tids1)
    oc = (oc0, oc1)
    ot = (ot0, ot1)
    sem_t = (sem_t0, sem_t1)
    sem_oc = (sem_oc0, sem_oc1)
    sem_ot = (sem_ot0, sem_ot1)

    pltpu.sync_copy(ctab_hbm.at[half], ctab_v)
    pltpu.sync_copy(ttab_hbm.at[half], ttab_v)

    def start_gathers(b, chunk_idx):
      start = tok_base + chunk_idx * CHUNK
      pltpu.sync_copy(seq_hbm.at[pl.ds(start, CHUNK)], ids[b])
      pltpu.async_copy(feat_hbm.at[ids[b]], tids[b], sem_t[b])

    def wait_gathers(b):
      pltpu.make_async_copy(feat_hbm.at[ids[b]], tids[b], sem_t[b]).wait()

    def out_dst(ref_hbm, chunk_idx):
      start = tok_base + chunk_idx * CHUNK
      return ref_hbm.at[pl.ds(start, CHUNK), pl.ds(c0, DH)]

    def start_writes(b, chunk_idx):
      pltpu.async_copy(oc[b], out_dst(out_cat_hbm, chunk_idx), sem_oc[b])
      pltpu.async_copy(ot[b], out_dst(out_tag_hbm, chunk_idx), sem_ot[b])

    def wait_writes(b, chunk_idx):
      pltpu.make_async_copy(oc[b], out_dst(out_cat_hbm, chunk_idx),
                            sem_oc[b]).wait()
      pltpu.make_async_copy(ot[b], out_dst(out_tag_hbm, chunk_idx),
                            sem_ot[b]).wait()

    iota16 = lax.iota(_i32, 16)

    def compute(b):
      @pl.loop(0, GROUPS)
      def group_loop(g):
        tok_idx = iota16 + g * 16
        rows = [
            plsc.load_gather(tids[b], [tok_idx, jnp.full((16,), s, _i32)])
            for s in range(TAG_LEN + 1)
        ]
        for c in range(DH):
          cvec = jnp.full((16,), c, _i32)
          catv = plsc.load_gather(ctab_v, [rows[TAG_LEN], cvec])
          plsc.store_scatter(oc[b], [tok_idx, cvec], catv)
          acc = plsc.load_gather(ttab_v, [rows[0], cvec])
          for s in range(1, TAG_LEN):
            acc = acc + plsc.load_gather(ttab_v, [rows[s], cvec])
          plsc.store_scatter(ot[b], [tok_idx, cvec], acc)

    start_gathers(0, 0)
    start_gathers(1, 1)

    @pl.loop(0, NCHUNK // 2)
    def chunk_loop(i):
      for b in (0, 1):
        cchunk = i * 2 + b
        wait_gathers(b)

        @pl.when(i >= 1)
        def _():
          wait_writes(b, cchunk - 2)

        compute(b)
        start_writes(b, cchunk)

        @pl.when(cchunk + 2 < NCHUNK)
        def _():
          start_gathers(b, cchunk + 2)

    wait_writes(0, NCHUNK - 2)
    wait_writes(1, NCHUNK - 1)

  return seq_emb


_SEQ_EMB = _build_kernel()


def kernel(item_seq, feat_category, feat_tags, cat_table, tag_table):
  feat_all = jnp.zeros((N_ITEMS_PAD := feat_category.shape[0], FEAT_W), _i32)
  feat_all = feat_all.at[:, :TAG_LEN].set(feat_tags)
  feat_all = feat_all.at[:, TAG_LEN].set(feat_category)
  ctab_halves = jnp.stack([cat_table[:, :DH], cat_table[:, DH:]])
  ttab_halves = jnp.stack([tag_table[:, :DH], tag_table[:, DH:]])
  out_cat, out_tag = _SEQ_EMB(item_seq.reshape(T), feat_all,
                              ctab_halves, ttab_halves)
  return out_cat.reshape(B, L, D), out_tag.reshape(B, L, D)


# speedup vs baseline: 6.0295x; 1.8888x over previous
import functools

import jax
import jax.numpy as jnp
from jax import lax
from jax.experimental import pallas as pl
from jax.experimental.pallas import tpu as pltpu
from jax.experimental.pallas import tpu_sc as plsc

B = 4096
L = 50
D = 64
T = B * L
CAT_V = 1000
TAG_V = 2000
TAG_LEN = 5
FEAT_W = 16
OUT_W = 32

NC = 2
NS = 16
NW = NC * NS
DH = D // 2
TOK_W = T // (NW // 2)
CHUNK = 128
NCHUNK = TOK_W // CHUNK
GROUPS = CHUNK // 16

_f32 = jnp.float32
_i32 = jnp.int32


def _build_kernel():
  mesh = plsc.VectorSubcoreMesh(core_axis_name="c", subcore_axis_name="s")

  @functools.partial(
      pl.kernel,
      out_type=(jax.ShapeDtypeStruct((T, D), _f32),
                jax.ShapeDtypeStruct((T, D), _f32)),
      mesh=mesh,
      compiler_params=pltpu.CompilerParams(use_tc_tiling_on_sc=False,
                                           needs_layout_passes=False),
      scratch_types=[
          pltpu.VMEM((DH * CAT_V,), _f32),
          pltpu.VMEM((DH * TAG_V,), _f32),
          pltpu.VMEM((CHUNK,), _i32),
          pltpu.VMEM((CHUNK,), _i32),
          pltpu.VMEM((CHUNK, FEAT_W), _i32),
          pltpu.VMEM((CHUNK, FEAT_W), _i32),
          pltpu.VMEM((CHUNK, OUT_W), _f32),
          pltpu.VMEM((CHUNK, OUT_W), _f32),
          pltpu.VMEM((CHUNK, OUT_W), _f32),
          pltpu.VMEM((CHUNK, OUT_W), _f32),
          pltpu.SemaphoreType.DMA,
          pltpu.SemaphoreType.DMA,
          pltpu.SemaphoreType.DMA,
          pltpu.SemaphoreType.DMA,
          pltpu.SemaphoreType.DMA,
          pltpu.SemaphoreType.DMA,
      ],
  )
  def seq_emb(seq_hbm, feat_hbm, ctab_hbm, ttab_hbm,
              out_cat_hbm, out_tag_hbm,
              ctab_v, ttab_v,
              ids0, ids1, tids0, tids1,
              oc0, oc1, ot0, ot1,
              sem_t0, sem_t1,
              sem_oc0, sem_oc1, sem_ot0, sem_ot1):
    wid = lax.axis_index("s") * NC + lax.axis_index("c")
    half = wid % 2
    tok_base = (wid // 2) * TOK_W
    c0 = half * DH

    ids = (ids0, ids1)
    tids = (tids0, tids1)
    oc = (oc0, oc1)
    ot = (ot0, ot1)
    sem_t = (sem_t0, sem_t1)
    sem_oc = (sem_oc0, sem_oc1)
    sem_ot = (sem_ot0, sem_ot1)

    pltpu.sync_copy(ctab_hbm.at[half], ctab_v)
    pltpu.sync_copy(ttab_hbm.at[half], ttab_v)

    def start_gathers(b, chunk_idx):
      start = tok_base + chunk_idx * CHUNK
      pltpu.sync_copy(seq_hbm.at[pl.ds(start, CHUNK)], ids[b])
      pltpu.async_copy(feat_hbm.at[ids[b]], tids[b], sem_t[b])

    def wait_gathers(b):
      pltpu.make_async_copy(feat_hbm.at[ids[b]], tids[b], sem_t[b]).wait()

    def out_dst(ref_hbm, chunk_idx):
      start = tok_base + chunk_idx * CHUNK
      return ref_hbm.at[pl.ds(start, CHUNK), pl.ds(c0, DH)]

    def start_writes(b, chunk_idx):
      pltpu.async_copy(oc[b], out_dst(out_cat_hbm, chunk_idx), sem_oc[b])
      pltpu.async_copy(ot[b], out_dst(out_tag_hbm, chunk_idx), sem_ot[b])

    def wait_writes(b, chunk_idx):
      pltpu.make_async_copy(oc[b], out_dst(out_cat_hbm, chunk_idx),
                            sem_oc[b]).wait()
      pltpu.make_async_copy(ot[b], out_dst(out_tag_hbm, chunk_idx),
                            sem_ot[b]).wait()

    iota16 = lax.iota(_i32, 16)

    def compute(b):
      @pl.loop(0, GROUPS)
      def group_loop(g):
        tok_idx = iota16 + g * 16
        rows = [
            plsc.load_gather(tids[b], [tok_idx, jnp.full((16,), s, _i32)])
            for s in range(TAG_LEN + 1)
        ]
        for c in range(DH):
          cvec = jnp.full((16,), c, _i32)
          catv = plsc.load_gather(ctab_v, [rows[TAG_LEN] + c * CAT_V])
          plsc.store_scatter(oc[b], [tok_idx, cvec], catv)
          t0 = plsc.load_gather(ttab_v, [rows[0] + c * TAG_V])
          t1 = plsc.load_gather(ttab_v, [rows[1] + c * TAG_V])
          t2 = plsc.load_gather(ttab_v, [rows[2] + c * TAG_V])
          t3 = plsc.load_gather(ttab_v, [rows[3] + c * TAG_V])
          t4 = plsc.load_gather(ttab_v, [rows[4] + c * TAG_V])
          acc = ((t0 + t1) + (t2 + t3)) + t4
          plsc.store_scatter(ot[b], [tok_idx, cvec], acc)

    start_gathers(0, 0)
    start_gathers(1, 1)

    @pl.loop(0, NCHUNK // 2)
    def chunk_loop(i):
      for b in (0, 1):
        cchunk = i * 2 + b
        wait_gathers(b)

        @pl.when(i >= 1)
        def _():
          wait_writes(b, cchunk - 2)

        compute(b)
        start_writes(b, cchunk)

        @pl.when(cchunk + 2 < NCHUNK)
        def _():
          start_gathers(b, cchunk + 2)

    wait_writes(0, NCHUNK - 2)
    wait_writes(1, NCHUNK - 1)

  return seq_emb


_SEQ_EMB = _build_kernel()


def kernel(item_seq, feat_category, feat_tags, cat_table, tag_table):
  n_items = feat_category.shape[0]
  feat_all = jnp.zeros((n_items, FEAT_W), _i32)
  feat_all = feat_all.at[:, :TAG_LEN].set(feat_tags)
  feat_all = feat_all.at[:, TAG_LEN].set(feat_category)
  ctab_t = cat_table.T.reshape(2, DH * CAT_V)
  ttab_t = tag_table.T.reshape(2, DH * TAG_V)
  out_cat, out_tag = _SEQ_EMB(item_seq.reshape(T), feat_all, ctab_t, ttab_t)
  return out_cat.reshape(B, L, D), out_tag.reshape(B, L, D)


# speedup vs baseline: 6.9260x; 1.1487x over previous
import functools

import jax
import jax.numpy as jnp
from jax import lax
from jax.experimental import pallas as pl
from jax.experimental.pallas import tpu as pltpu
from jax.experimental.pallas import tpu_sc as plsc

B = 4096
L = 50
D = 64
T = B * L
CAT_V = 1000
TAG_V = 2000
TAG_LEN = 5
FEAT_W = 16
OUT_W = 33

NC = 2
NS = 16
NW = NC * NS
DH = D // 2
TOK_W = T // (NW // 2)
CHUNK = 128
NCHUNK = TOK_W // CHUNK
GROUPS = CHUNK // 16

_f32 = jnp.float32
_i32 = jnp.int32


def _build_kernel():
  mesh = plsc.VectorSubcoreMesh(core_axis_name="c", subcore_axis_name="s")

  @functools.partial(
      pl.kernel,
      out_type=(jax.ShapeDtypeStruct((T, D), _f32),
                jax.ShapeDtypeStruct((T, D), _f32)),
      mesh=mesh,
      compiler_params=pltpu.CompilerParams(use_tc_tiling_on_sc=False,
                                           needs_layout_passes=False),
      scratch_types=[
          pltpu.VMEM((DH * CAT_V,), _f32),
          pltpu.VMEM((DH * TAG_V,), _f32),
          pltpu.VMEM((CHUNK,), _i32),
          pltpu.VMEM((CHUNK,), _i32),
          pltpu.VMEM((CHUNK, FEAT_W), _i32),
          pltpu.VMEM((CHUNK, FEAT_W), _i32),
          pltpu.VMEM((CHUNK, OUT_W), _f32),
          pltpu.VMEM((CHUNK, OUT_W), _f32),
          pltpu.VMEM((CHUNK, OUT_W), _f32),
          pltpu.VMEM((CHUNK, OUT_W), _f32),
          pltpu.SemaphoreType.DMA,
          pltpu.SemaphoreType.DMA,
          pltpu.SemaphoreType.DMA,
          pltpu.SemaphoreType.DMA,
          pltpu.SemaphoreType.DMA,
          pltpu.SemaphoreType.DMA,
      ],
  )
  def seq_emb(seq_hbm, feat_hbm, ctab_hbm, ttab_hbm,
              out_cat_hbm, out_tag_hbm,
              ctab_v, ttab_v,
              ids0, ids1, tids0, tids1,
              oc0, oc1, ot0, ot1,
              sem_t0, sem_t1,
              sem_oc0, sem_oc1, sem_ot0, sem_ot1):
    wid = lax.axis_index("s") * NC + lax.axis_index("c")
    half = wid % 2
    tok_base = (wid // 2) * TOK_W
    c0 = half * DH

    ids = (ids0, ids1)
    tids = (tids0, tids1)
    oc = (oc0, oc1)
    ot = (ot0, ot1)
    sem_t = (sem_t0, sem_t1)
    sem_oc = (sem_oc0, sem_oc1)
    sem_ot = (sem_ot0, sem_ot1)

    pltpu.sync_copy(ctab_hbm.at[half], ctab_v)
    pltpu.sync_copy(ttab_hbm.at[half], ttab_v)

    def start_gathers(b, chunk_idx):
      start = tok_base + chunk_idx * CHUNK
      pltpu.sync_copy(seq_hbm.at[pl.ds(start, CHUNK)], ids[b])
      pltpu.async_copy(feat_hbm.at[ids[b]], tids[b], sem_t[b])

    def wait_gathers(b):
      pltpu.make_async_copy(feat_hbm.at[ids[b]], tids[b], sem_t[b]).wait()

    def out_dst(ref_hbm, chunk_idx):
      start = tok_base + chunk_idx * CHUNK
      return ref_hbm.at[pl.ds(start, CHUNK), pl.ds(c0, DH)]

    def start_writes(b, chunk_idx):
      pltpu.async_copy(oc[b].at[:, pl.ds(0, DH)],
                       out_dst(out_cat_hbm, chunk_idx), sem_oc[b])
      pltpu.async_copy(ot[b].at[:, pl.ds(0, DH)],
                       out_dst(out_tag_hbm, chunk_idx), sem_ot[b])

    def wait_writes(b, chunk_idx):
      pltpu.make_async_copy(oc[b].at[:, pl.ds(0, DH)],
                            out_dst(out_cat_hbm, chunk_idx),
                            sem_oc[b]).wait()
      pltpu.make_async_copy(ot[b].at[:, pl.ds(0, DH)],
                            out_dst(out_tag_hbm, chunk_idx),
                            sem_ot[b]).wait()

    iota16 = lax.iota(_i32, 16)

    def compute(b):
      @pl.loop(0, GROUPS)
      def group_loop(g):
        tok_idx = iota16 + g * 16
        rows = [
            plsc.load_gather(tids[b], [tok_idx, jnp.full((16,), s, _i32)])
            for s in range(TAG_LEN + 1)
        ]
        for c in range(DH):
          cvec = jnp.full((16,), c, _i32)
          catv = plsc.load_gather(ctab_v, [rows[TAG_LEN] + c * CAT_V])
          plsc.store_scatter(oc[b], [tok_idx, cvec], catv)
          t0 = plsc.load_gather(ttab_v, [rows[0] + c * TAG_V])
          t1 = plsc.load_gather(ttab_v, [rows[1] + c * TAG_V])
          t2 = plsc.load_gather(ttab_v, [rows[2] + c * TAG_V])
          t3 = plsc.load_gather(ttab_v, [rows[3] + c * TAG_V])
          t4 = plsc.load_gather(ttab_v, [rows[4] + c * TAG_V])
          acc = ((t0 + t1) + (t2 + t3)) + t4
          plsc.store_scatter(ot[b], [tok_idx, cvec], acc)

    start_gathers(0, 0)
    start_gathers(1, 1)

    @pl.loop(0, NCHUNK // 2)
    def chunk_loop(i):
      for b in (0, 1):
        cchunk = i * 2 + b
        wait_gathers(b)

        @pl.when(i >= 1)
        def _():
          wait_writes(b, cchunk - 2)

        compute(b)
        start_writes(b, cchunk)

        @pl.when(cchunk + 2 < NCHUNK)
        def _():
          start_gathers(b, cchunk + 2)

    wait_writes(0, NCHUNK - 2)
    wait_writes(1, NCHUNK - 1)

  return seq_emb


_SEQ_EMB = _build_kernel()


def kernel(item_seq, feat_category, feat_tags, cat_table, tag_table):
  n_items = feat_category.shape[0]
  feat_all = jnp.zeros((n_items, FEAT_W), _i32)
  feat_all = feat_all.at[:, :TAG_LEN].set(feat_tags)
  feat_all = feat_all.at[:, TAG_LEN].set(feat_category)
  ctab_t = cat_table.T.reshape(2, DH * CAT_V)
  ttab_t = tag_table.T.reshape(2, DH * TAG_V)
  out_cat, out_tag = _SEQ_EMB(item_seq.reshape(T), feat_all, ctab_t, ttab_t)
  return out_cat.reshape(B, L, D), out_tag.reshape(B, L, D)


# speedup vs baseline: 9.0428x; 1.3056x over previous
import functools

import jax
import jax.numpy as jnp
from jax import lax
from jax.experimental import pallas as pl
from jax.experimental.pallas import tpu as pltpu
from jax.experimental.pallas import tpu_sc as plsc

B = 4096
L = 50
D = 64
T = B * L
CAT_V = 1000
TAG_V = 2000
TAG_LEN = 5
FEAT_W = 16
OUT_W = 33

NC = 2
NS = 16
NW = NC * NS
DH = D // 2
TOK_W = T // (NW // 2)
CHUNK = 128
NCHUNK = TOK_W // CHUNK
GROUPS = CHUNK // 16

_f32 = jnp.float32
_i32 = jnp.int32


def _build_kernel():
  mesh = plsc.VectorSubcoreMesh(core_axis_name="c", subcore_axis_name="s")

  @functools.partial(
      pl.kernel,
      out_type=(jax.ShapeDtypeStruct((T, D), _f32),
                jax.ShapeDtypeStruct((T, D), _f32)),
      mesh=mesh,
      compiler_params=pltpu.CompilerParams(use_tc_tiling_on_sc=False,
                                           needs_layout_passes=False),
      scratch_types=[
          pltpu.VMEM((DH * CAT_V,), _f32),
          pltpu.VMEM((DH * TAG_V,), _f32),
          pltpu.VMEM((CHUNK,), _i32),
          pltpu.VMEM((CHUNK,), _i32),
          pltpu.VMEM((CHUNK, FEAT_W), _i32),
          pltpu.VMEM((CHUNK, FEAT_W), _i32),
          pltpu.VMEM((CHUNK, OUT_W), _f32),
          pltpu.VMEM((CHUNK, OUT_W), _f32),
          pltpu.VMEM((CHUNK, OUT_W), _f32),
          pltpu.VMEM((CHUNK, OUT_W), _f32),
          pltpu.SemaphoreType.DMA,
          pltpu.SemaphoreType.DMA,
          pltpu.SemaphoreType.DMA,
          pltpu.SemaphoreType.DMA,
          pltpu.SemaphoreType.DMA,
          pltpu.SemaphoreType.DMA,
      ],
  )
  def seq_emb(seq_hbm, feat_hbm, ctab_hbm, ttab_hbm,
              out_cat_hbm, out_tag_hbm,
              ctab_v, ttab_v,
              ids0, ids1, tids0, tids1,
              oc0, oc1, ot0, ot1,
              sem_t0, sem_t1,
              sem_oc0, sem_oc1, sem_ot0, sem_ot1):
    wid = lax.axis_index("s") * NC + lax.axis_index("c")
    half = wid % 2
    tok_base = (wid // 2) * TOK_W
    c0 = half * DH

    ids = (ids0, ids1)
    tids = (tids0, tids1)
    oc = (oc0, oc1)
    ot = (ot0, ot1)
    sem_t = (sem_t0, sem_t1)
    sem_oc = (sem_oc0, sem_oc1)
    sem_ot = (sem_ot0, sem_ot1)

    pltpu.sync_copy(ctab_hbm.at[half], ctab_v)
    pltpu.sync_copy(ttab_hbm.at[half], ttab_v)

    def start_gathers(b, chunk_idx):
      start = tok_base + chunk_idx * CHUNK
      pltpu.sync_copy(seq_hbm.at[pl.ds(start, CHUNK)], ids[b])
      pltpu.async_copy(feat_hbm.at[ids[b]], tids[b], sem_t[b])

    def wait_gathers(b):
      pltpu.make_async_copy(feat_hbm.at[ids[b]], tids[b], sem_t[b]).wait()

    def out_dst(ref_hbm, chunk_idx):
      start = tok_base + chunk_idx * CHUNK
      return ref_hbm.at[pl.ds(start, CHUNK), pl.ds(c0, DH)]

    def start_writes(b, chunk_idx):
      pltpu.async_copy(oc[b].at[:, pl.ds(0, DH)],
                       out_dst(out_cat_hbm, chunk_idx), sem_oc[b])
      pltpu.async_copy(ot[b].at[:, pl.ds(0, DH)],
                       out_dst(out_tag_hbm, chunk_idx), sem_ot[b])

    def wait_writes(b, chunk_idx):
      pltpu.make_async_copy(oc[b].at[:, pl.ds(0, DH)],
                            out_dst(out_cat_hbm, chunk_idx),
                            sem_oc[b]).wait()
      pltpu.make_async_copy(ot[b].at[:, pl.ds(0, DH)],
                            out_dst(out_tag_hbm, chunk_idx),
                            sem_ot[b]).wait()

    iota16 = lax.iota(_i32, 16)

    def compute(b):
      @pl.loop(0, GROUPS)
      def group_loop(g):
        tok_idx = iota16 + g * 16
        rows = [
            plsc.load_gather(tids[b], [tok_idx, jnp.full((16,), s, _i32)])
            for s in range(TAG_LEN + 1)
        ]
        for c in range(DH):
          cvec = jnp.full((16,), c, _i32)
          catv = plsc.load_gather(ctab_v, [rows[TAG_LEN] + c * CAT_V])
          plsc.store_scatter(oc[b], [tok_idx, cvec], catv)
          t0 = plsc.load_gather(ttab_v, [rows[0] + c * TAG_V])
          t1 = plsc.load_gather(ttab_v, [rows[1] + c * TAG_V])
          t2 = plsc.load_gather(ttab_v, [rows[2] + c * TAG_V])
          t3 = plsc.load_gather(ttab_v, [rows[3] + c * TAG_V])
          t4 = plsc.load_gather(ttab_v, [rows[4] + c * TAG_V])
          acc = ((t0 + t1) + (t2 + t3)) + t4
          plsc.store_scatter(ot[b], [tok_idx, cvec], acc)

    start_gathers(0, 0)
    start_gathers(1, 1)

    @pl.loop(0, NCHUNK // 2)
    def chunk_loop(i):
      for b in (0, 1):
        cchunk = i * 2 + b
        wait_gathers(b)

        @pl.when(i >= 1)
        def _():
          wait_writes(b, cchunk - 2)

        compute(b)
        start_writes(b, cchunk)

        @pl.when(cchunk + 2 < NCHUNK)
        def _():
          start_gathers(b, cchunk + 2)

    wait_writes(0, NCHUNK - 2)
    wait_writes(1, NCHUNK - 1)

  return seq_emb


_SEQ_EMB = _build_kernel()


def kernel(item_seq, feat_category, feat_tags, cat_table, tag_table):
  n_items = feat_category.shape[0]
  feat_all = jnp.concatenate(
      [feat_tags, feat_category[:, None],
       jnp.zeros((n_items, FEAT_W - TAG_LEN - 1), _i32)], axis=1)
  ctab_t = cat_table.T.reshape(2, DH * CAT_V)
  ttab_t = tag_table.T.reshape(2, DH * TAG_V)
  out_cat, out_tag = _SEQ_EMB(item_seq.reshape(T), feat_all, ctab_t, ttab_t)
  return out_cat.reshape(B, L, D), out_tag.reshape(B, L, D)


# speedup vs baseline: 11.5432x; 1.2765x over previous
import functools

import jax
import jax.numpy as jnp
from jax import lax
from jax.experimental import pallas as pl
from jax.experimental.pallas import tpu as pltpu
from jax.experimental.pallas import tpu_sc as plsc

B = 4096
L = 50
D = 64
T = B * L
CAT_V = 1000
TAG_V = 2000
TAG_LEN = 5
FEAT_W = 16
OUT_W = 33

NC = 2
NS = 16
NW = NC * NS
DH = D // 2
TOK_W = T // (NW // 2)
CHUNK = 128
NCHUNK = TOK_W // CHUNK
GROUPS = CHUNK // 16

_f32 = jnp.float32
_i32 = jnp.int32


def _build_kernel():
  mesh = plsc.VectorSubcoreMesh(core_axis_name="c", subcore_axis_name="s")

  @functools.partial(
      pl.kernel,
      out_type=(jax.ShapeDtypeStruct((T, D), _f32),
                jax.ShapeDtypeStruct((T, D), _f32)),
      mesh=mesh,
      compiler_params=pltpu.CompilerParams(use_tc_tiling_on_sc=False,
                                           needs_layout_passes=False),
      scratch_types=[
          pltpu.VMEM((DH * TAG_V,), _f32),
          pltpu.VMEM((CHUNK,), _i32),
          pltpu.VMEM((CHUNK,), _i32),
          pltpu.VMEM((CHUNK, FEAT_W), _i32),
          pltpu.VMEM((CHUNK, FEAT_W), _i32),
          pltpu.VMEM((CHUNK,), _i32),
          pltpu.VMEM((CHUNK,), _i32),
          pltpu.VMEM((CHUNK, D), _f32),
          pltpu.VMEM((CHUNK, D), _f32),
          pltpu.VMEM((CHUNK, OUT_W), _f32),
          pltpu.VMEM((CHUNK, OUT_W), _f32),
          pltpu.SemaphoreType.DMA,
          pltpu.SemaphoreType.DMA,
          pltpu.SemaphoreType.DMA,
          pltpu.SemaphoreType.DMA,
          pltpu.SemaphoreType.DMA,
          pltpu.SemaphoreType.DMA,
          pltpu.SemaphoreType.DMA,
          pltpu.SemaphoreType.DMA,
      ],
  )
  def seq_emb(seq_hbm, feat_hbm, ctab_hbm, ttab_hbm,
              out_cat_hbm, out_tag_hbm,
              ttab_v,
              ids0, ids1, tids0, tids1, cids0, cids1,
              cs0, cs1, ot0, ot1,
              sem_t0, sem_t1, sem_g0, sem_g1,
              sem_cw0, sem_cw1, sem_tw0, sem_tw1):
    wid = lax.axis_index("s") * NC + lax.axis_index("c")
    half = wid % 2
    tok_base = (wid // 2) * TOK_W
    c0 = half * DH

    ids = (ids0, ids1)
    tids = (tids0, tids1)
    cids = (cids0, cids1)
    cs = (cs0, cs1)
    ot = (ot0, ot1)
    sem_t = (sem_t0, sem_t1)
    sem_g = (sem_g0, sem_g1)
    sem_cw = (sem_cw0, sem_cw1)
    sem_tw = (sem_tw0, sem_tw1)

    pltpu.sync_copy(ttab_hbm.at[half], ttab_v)

    def start_feat(b, chunk_idx):
      start = tok_base + chunk_idx * CHUNK
      pltpu.sync_copy(seq_hbm.at[pl.ds(start, CHUNK)], ids[b])
      pltpu.async_copy(feat_hbm.at[ids[b]], tids[b], sem_t[b])

    def wait_feat(b):
      pltpu.make_async_copy(feat_hbm.at[ids[b]], tids[b], sem_t[b]).wait()

    def tag_dst(chunk_idx):
      start = tok_base + chunk_idx * CHUNK
      return out_tag_hbm.at[pl.ds(start, CHUNK), pl.ds(c0, DH)]

    def cat_dst(chunk_idx):
      start = tok_base + chunk_idx * CHUNK
      return out_cat_hbm.at[pl.ds(start, CHUNK), :]

    def start_tagw(b, chunk_idx):
      pltpu.async_copy(ot[b].at[:, pl.ds(0, DH)], tag_dst(chunk_idx),
                       sem_tw[b])

    def wait_tagw(b, chunk_idx):
      pltpu.make_async_copy(ot[b].at[:, pl.ds(0, DH)], tag_dst(chunk_idx),
                            sem_tw[b]).wait()

    def start_catg(b):
      pltpu.async_copy(ctab_hbm.at[cids[b]], cs[b], sem_g[b])

    def wait_catg(b):
      pltpu.make_async_copy(ctab_hbm.at[cids[b]], cs[b], sem_g[b]).wait()

    def start_catw(b, chunk_idx):
      pltpu.async_copy(cs[b], cat_dst(chunk_idx), sem_cw[b])

    def wait_catw(b, chunk_idx):
      pltpu.make_async_copy(cs[b], cat_dst(chunk_idx), sem_cw[b]).wait()

    iota16 = lax.iota(_i32, 16)

    def extract_cids(b):
      @pl.loop(0, GROUPS)
      def _(g):
        tok_idx = iota16 + g * 16
        cv = plsc.load_gather(tids[b],
                              [tok_idx, jnp.full((16,), TAG_LEN, _i32)])
        cids[b][pl.ds(g * 16, 16)] = cv

    def compute_tags(b):
      @pl.loop(0, GROUPS)
      def group_loop(g):
        tok_idx = iota16 + g * 16
        rows = [
            plsc.load_gather(tids[b], [tok_idx, (iota16 + s) % TAG_LEN])
            for s in range(TAG_LEN)
        ]
        for c in range(DH):
          cvec = jnp.full((16,), c, _i32)
          t0 = plsc.load_gather(ttab_v, [rows[0] + c * TAG_V])
          t1 = plsc.load_gather(ttab_v, [rows[1] + c * TAG_V])
          t2 = plsc.load_gather(ttab_v, [rows[2] + c * TAG_V])
          t3 = plsc.load_gather(ttab_v, [rows[3] + c * TAG_V])
          t4 = plsc.load_gather(ttab_v, [rows[4] + c * TAG_V])
          acc = ((t0 + t1) + (t2 + t3)) + t4
          plsc.store_scatter(ot[b], [tok_idx, cvec], acc)

    start_feat(0, 0)
    start_feat(1, 1)

    @pl.loop(0, NCHUNK // 2)
    def chunk_loop(i):
      for b in (0, 1):
        cchunk = i * 2 + b
        wait_feat(b)

        @pl.when(half == 0)
        def _():
          @pl.when(i >= 1)
          def _():
            wait_catw(b, cchunk - 2)

          extract_cids(b)
          start_catg(b)

        @pl.when(i >= 1)
        def _():
          wait_tagw(b, cchunk - 2)

        compute_tags(b)
        start_tagw(b, cchunk)

        @pl.when(half == 0)
        def _():
          wait_catg(b)
          start_catw(b, cchunk)

        @pl.when(cchunk + 2 < NCHUNK)
        def _():
          start_feat(b, cchunk + 2)

    wait_tagw(0, NCHUNK - 2)
    wait_tagw(1, NCHUNK - 1)

    @pl.when(half == 0)
    def _():
      wait_catw(0, NCHUNK - 2)
      wait_catw(1, NCHUNK - 1)

  return seq_emb


_SEQ_EMB = _build_kernel()


def kernel(item_seq, feat_category, feat_tags, cat_table, tag_table):
  n_items = feat_category.shape[0]
  feat_all = jnp.concatenate(
      [feat_tags, feat_category[:, None],
       jnp.zeros((n_items, FEAT_W - TAG_LEN - 1), _i32)], axis=1)
  ttab_t = tag_table.T.reshape(2, DH * TAG_V)
  out_cat, out_tag = _SEQ_EMB(item_seq.reshape(T), feat_all, cat_table,
                              ttab_t)
  return out_cat.reshape(B, L, D), out_tag.reshape(B, L, D)


# speedup vs baseline: 13.7515x; 1.1913x over previous
import functools

import jax
import jax.numpy as jnp
from jax import lax
from jax.experimental import pallas as pl
from jax.experimental.pallas import tpu as pltpu
from jax.experimental.pallas import tpu_sc as plsc

B = 4096
L = 50
D = 64
T = B * L
CAT_V = 1000
TAG_V = 2000
TAG_LEN = 5
FEAT_W = 16
OUT_W = 33

NC = 2
NS = 16
NW = NC * NS
DH = D // 2
TOK_W = T // (NW // 2)
CHUNK = 128
NCHUNK = TOK_W // CHUNK
GROUPS = CHUNK // 16

_f32 = jnp.float32
_i32 = jnp.int32


def _build_kernel():
  mesh = plsc.VectorSubcoreMesh(core_axis_name="c", subcore_axis_name="s")

  @functools.partial(
      pl.kernel,
      out_type=(jax.ShapeDtypeStruct((T, D), _f32),
                jax.ShapeDtypeStruct((T, D), _f32)),
      mesh=mesh,
      compiler_params=pltpu.CompilerParams(use_tc_tiling_on_sc=False,
                                           needs_layout_passes=False),
      scratch_types=[
          pltpu.VMEM((DH // 2 * TAG_V,), _i32),
          pltpu.VMEM((CHUNK,), _i32),
          pltpu.VMEM((CHUNK,), _i32),
          pltpu.VMEM((CHUNK, FEAT_W), _i32),
          pltpu.VMEM((CHUNK, FEAT_W), _i32),
          pltpu.VMEM((CHUNK,), _i32),
          pltpu.VMEM((CHUNK,), _i32),
          pltpu.VMEM((CHUNK, D), _f32),
          pltpu.VMEM((CHUNK, D), _f32),
          pltpu.VMEM((CHUNK, OUT_W), _f32),
          pltpu.VMEM((CHUNK, OUT_W), _f32),
          pltpu.SemaphoreType.DMA,
          pltpu.SemaphoreType.DMA,
          pltpu.SemaphoreType.DMA,
          pltpu.SemaphoreType.DMA,
          pltpu.SemaphoreType.DMA,
          pltpu.SemaphoreType.DMA,
          pltpu.SemaphoreType.DMA,
          pltpu.SemaphoreType.DMA,
      ],
  )
  def seq_emb(seq_hbm, feat_hbm, ctab_hbm, ttab_hbm,
              out_cat_hbm, out_tag_hbm,
              ttab_v,
              ids0, ids1, tids0, tids1, cids0, cids1,
              cs0, cs1, ot0, ot1,
              sem_t0, sem_t1, sem_g0, sem_g1,
              sem_cw0, sem_cw1, sem_tw0, sem_tw1):
    wid = lax.axis_index("s") * NC + lax.axis_index("c")
    half = wid % 2
    tok_base = (wid // 2) * TOK_W
    c0 = half * DH

    ids = (ids0, ids1)
    tids = (tids0, tids1)
    cids = (cids0, cids1)
    cs = (cs0, cs1)
    ot = (ot0, ot1)
    sem_t = (sem_t0, sem_t1)
    sem_g = (sem_g0, sem_g1)
    sem_cw = (sem_cw0, sem_cw1)
    sem_tw = (sem_tw0, sem_tw1)

    pltpu.sync_copy(ttab_hbm.at[half], ttab_v)

    def start_feat(b, chunk_idx):
      start = tok_base + chunk_idx * CHUNK
      pltpu.sync_copy(seq_hbm.at[pl.ds(start, CHUNK)], ids[b])
      pltpu.async_copy(feat_hbm.at[ids[b]], tids[b], sem_t[b])

    def wait_feat(b):
      pltpu.make_async_copy(feat_hbm.at[ids[b]], tids[b], sem_t[b]).wait()

    def tag_dst(chunk_idx):
      start = tok_base + chunk_idx * CHUNK
      return out_tag_hbm.at[pl.ds(start, CHUNK), pl.ds(c0, DH)]

    def cat_dst(chunk_idx):
      start = tok_base + chunk_idx * CHUNK
      return out_cat_hbm.at[pl.ds(start, CHUNK), :]

    def start_tagw(b, chunk_idx):
      pltpu.async_copy(ot[b].at[:, pl.ds(0, DH)], tag_dst(chunk_idx),
                       sem_tw[b])

    def wait_tagw(b, chunk_idx):
      pltpu.make_async_copy(ot[b].at[:, pl.ds(0, DH)], tag_dst(chunk_idx),
                            sem_tw[b]).wait()

    def start_catg(b):
      pltpu.async_copy(ctab_hbm.at[cids[b]], cs[b], sem_g[b])

    def wait_catg(b):
      pltpu.make_async_copy(ctab_hbm.at[cids[b]], cs[b], sem_g[b]).wait()

    def start_catw(b, chunk_idx):
      pltpu.async_copy(cs[b], cat_dst(chunk_idx), sem_cw[b])

    def wait_catw(b, chunk_idx):
      pltpu.make_async_copy(cs[b], cat_dst(chunk_idx), sem_cw[b]).wait()

    iota16 = lax.iota(_i32, 16)

    def extract_cids(b):
      @pl.loop(0, GROUPS)
      def _(g):
        tok_idx = iota16 + g * 16
        cv = plsc.load_gather(tids[b],
                              [tok_idx, jnp.full((16,), TAG_LEN, _i32)])
        cids[b][pl.ds(g * 16, 16)] = cv

    def compute_tags(b):
      @pl.loop(0, GROUPS)
      def group_loop(g):
        tok_idx = iota16 + g * 16
        rows = [
            plsc.load_gather(tids[b], [tok_idx, (iota16 + s) % TAG_LEN])
            for s in range(TAG_LEN)
        ]
        for p in range(DH // 2):
          acc_e = None
          acc_o = None
          for s in range(TAG_LEN):
            w = plsc.load_gather(ttab_v, [rows[s] + p * TAG_V])
            pair = plsc.bitcast(w, jnp.bfloat16)
            lo, hi = plsc.unpack(pair, format=plsc.PackFormat.INTERLEAVED)
            acc_e = lo if acc_e is None else acc_e + lo
            acc_o = hi if acc_o is None else acc_o + hi
          plsc.store_scatter(ot[b], [tok_idx, jnp.full((16,), 2 * p, _i32)],
                             acc_e)
          plsc.store_scatter(ot[b],
                             [tok_idx, jnp.full((16,), 2 * p + 1, _i32)],
                             acc_o)

    start_feat(0, 0)
    start_feat(1, 1)

    @pl.loop(0, NCHUNK // 2)
    def chunk_loop(i):
      for b in (0, 1):
        cchunk = i * 2 + b
        wait_feat(b)

        @pl.when(half == 0)
        def _():
          @pl.when(i >= 1)
          def _():
            wait_catw(b, cchunk - 2)

          extract_cids(b)
          start_catg(b)

        @pl.when(i >= 1)
        def _():
          wait_tagw(b, cchunk - 2)

        compute_tags(b)
        start_tagw(b, cchunk)

        @pl.when(half == 0)
        def _():
          wait_catg(b)
          start_catw(b, cchunk)

        @pl.when(cchunk + 2 < NCHUNK)
        def _():
          start_feat(b, cchunk + 2)

    wait_tagw(0, NCHUNK - 2)
    wait_tagw(1, NCHUNK - 1)

    @pl.when(half == 0)
    def _():
      wait_catw(0, NCHUNK - 2)
      wait_catw(1, NCHUNK - 1)

  return seq_emb


_SEQ_EMB = _build_kernel()


def kernel(item_seq, feat_category, feat_tags, cat_table, tag_table):
  n_items = feat_category.shape[0]
  feat_all = jnp.concatenate(
      [feat_tags, feat_category[:, None],
       jnp.zeros((n_items, FEAT_W - TAG_LEN - 1), _i32)], axis=1)
  tt_pairs = jax.lax.bitcast_convert_type(
      tag_table.astype(jnp.bfloat16).reshape(TAG_V, DH, 2), _i32)
  ttab_t = tt_pairs.T.reshape(2, DH // 2 * TAG_V)
  out_cat, out_tag = _SEQ_EMB(item_seq.reshape(T), feat_all, cat_table,
                              ttab_t)
  return out_cat.reshape(B, L, D), out_tag.reshape(B, L, D)


# speedup vs baseline: 15.4696x; 1.1249x over previous
import functools

import jax
import jax.numpy as jnp
from jax import lax
from jax.experimental import pallas as pl
from jax.experimental.pallas import tpu as pltpu
from jax.experimental.pallas import tpu_sc as plsc

B = 4096
L = 50
D = 64
T = B * L
CAT_V = 1000
TAG_V = 2000
TAG_LEN = 5
FEAT_W = 16
OUT_W = 33

NC = 2
NS = 16
NW = NC * NS
DH = D // 2
TOK_W = T // (NW // 2)
CHUNK = 128
NCHUNK = TOK_W // CHUNK
GROUPS = CHUNK // 16

_f32 = jnp.float32
_i32 = jnp.int32


def _build_kernel():
  mesh = plsc.VectorSubcoreMesh(core_axis_name="c", subcore_axis_name="s")

  @functools.partial(
      pl.kernel,
      out_type=(jax.ShapeDtypeStruct((T, D), _f32),
                jax.ShapeDtypeStruct((T, D), _f32)),
      mesh=mesh,
      compiler_params=pltpu.CompilerParams(use_tc_tiling_on_sc=False,
                                           needs_layout_passes=False),
      scratch_types=[
          pltpu.VMEM((DH // 2 * TAG_V,), _i32),
          pltpu.VMEM((CHUNK,), _i32),
          pltpu.VMEM((CHUNK,), _i32),
          pltpu.VMEM((CHUNK, FEAT_W), _i32),
          pltpu.VMEM((CHUNK, FEAT_W), _i32),
          pltpu.VMEM((CHUNK,), _i32),
          pltpu.VMEM((CHUNK,), _i32),
          pltpu.VMEM((CHUNK, D), _f32),
          pltpu.VMEM((CHUNK, D), _f32),
          pltpu.VMEM((CHUNK, OUT_W), _f32),
          pltpu.VMEM((CHUNK, OUT_W), _f32),
          pltpu.SemaphoreType.DMA,
          pltpu.SemaphoreType.DMA,
          pltpu.SemaphoreType.DMA,
          pltpu.SemaphoreType.DMA,
          pltpu.SemaphoreType.DMA,
          pltpu.SemaphoreType.DMA,
          pltpu.SemaphoreType.DMA,
          pltpu.SemaphoreType.DMA,
      ],
  )
  def seq_emb(seq_hbm, feat_hbm, ctab_hbm, ttab_hbm,
              out_cat_hbm, out_tag_hbm,
              ttab_v,
              ids0, ids1, tids0, tids1, cids0, cids1,
              cs0, cs1, ot0, ot1,
              sem_t0, sem_t1, sem_g0, sem_g1,
              sem_cw0, sem_cw1, sem_tw0, sem_tw1):
    wid = lax.axis_index("s") * NC + lax.axis_index("c")
    half = wid % 2
    tok_base = (wid // 2) * TOK_W
    c0 = half * DH

    ids = (ids0, ids1)
    tids = (tids0, tids1)
    cids = (cids0, cids1)
    cs = (cs0, cs1)
    ot = (ot0, ot1)
    sem_t = (sem_t0, sem_t1)
    sem_g = (sem_g0, sem_g1)
    sem_cw = (sem_cw0, sem_cw1)
    sem_tw = (sem_tw0, sem_tw1)

    pltpu.sync_copy(ttab_hbm.at[half], ttab_v)

    def start_feat(b, chunk_idx):
      start = tok_base + chunk_idx * CHUNK
      pltpu.sync_copy(seq_hbm.at[pl.ds(start, CHUNK)], ids[b])
      pltpu.async_copy(feat_hbm.at[ids[b]], tids[b], sem_t[b])

    def wait_feat(b):
      pltpu.make_async_copy(feat_hbm.at[ids[b]], tids[b], sem_t[b]).wait()

    def tag_dst(chunk_idx):
      start = tok_base + chunk_idx * CHUNK
      return out_tag_hbm.at[pl.ds(start, CHUNK), pl.ds(c0, DH)]

    def cat_dst(chunk_idx):
      start = tok_base + chunk_idx * CHUNK
      return out_cat_hbm.at[pl.ds(start, CHUNK), :]

    def start_tagw(b, chunk_idx):
      pltpu.async_copy(ot[b].at[:, pl.ds(0, DH)], tag_dst(chunk_idx),
                       sem_tw[b])

    def wait_tagw(b, chunk_idx):
      pltpu.make_async_copy(ot[b].at[:, pl.ds(0, DH)], tag_dst(chunk_idx),
                            sem_tw[b]).wait()

    def start_catg(b):
      pltpu.async_copy(ctab_hbm.at[cids[b]], cs[b], sem_g[b])

    def wait_catg(b):
      pltpu.make_async_copy(ctab_hbm.at[cids[b]], cs[b], sem_g[b]).wait()

    def start_catw(b, chunk_idx):
      pltpu.async_copy(cs[b], cat_dst(chunk_idx), sem_cw[b])

    def wait_catw(b, chunk_idx):
      pltpu.make_async_copy(cs[b], cat_dst(chunk_idx), sem_cw[b]).wait()

    iota16 = lax.iota(_i32, 16)

    def extract_cids(b):
      @pl.loop(0, GROUPS)
      def _(g):
        tok_idx = iota16 + g * 16
        cv = plsc.load_gather(tids[b],
                              [tok_idx, jnp.full((16,), TAG_LEN, _i32)])
        cids[b][pl.ds(g * 16, 16)] = cv

    def compute_tags(b):
      @plsc.parallel_loop(0, GROUPS, unroll=2)
      def group_loop(g):
        tok_idx = iota16 + g * 16
        rows = [
            plsc.load_gather(tids[b], [tok_idx, (iota16 + s) % TAG_LEN])
            for s in range(TAG_LEN)
        ]
        for p in range(DH // 2):
          acc_e = None
          acc_o = None
          for s in range(TAG_LEN):
            w = plsc.load_gather(ttab_v, [rows[s] + p * TAG_V])
            pair = plsc.bitcast(w, jnp.bfloat16)
            lo, hi = plsc.unpack(pair, format=plsc.PackFormat.INTERLEAVED)
            acc_e = lo if acc_e is None else acc_e + lo
            acc_o = hi if acc_o is None else acc_o + hi
          plsc.store_scatter(ot[b], [tok_idx, jnp.full((16,), 2 * p, _i32)],
                             acc_e)
          plsc.store_scatter(ot[b],
                             [tok_idx, jnp.full((16,), 2 * p + 1, _i32)],
                             acc_o)

    start_feat(0, 0)
    start_feat(1, 1)

    @pl.loop(0, NCHUNK // 2)
    def chunk_loop(i):
      for b in (0, 1):
        cchunk = i * 2 + b
        wait_feat(b)

        @pl.when(half == 0)
        def _():
          @pl.when(i >= 1)
          def _():
            wait_catw(b, cchunk - 2)

          extract_cids(b)
          start_catg(b)

        @pl.when(i >= 1)
        def _():
          wait_tagw(b, cchunk - 2)

        compute_tags(b)
        start_tagw(b, cchunk)

        @pl.when(half == 0)
        def _():
          wait_catg(b)
          start_catw(b, cchunk)

        @pl.when(cchunk + 2 < NCHUNK)
        def _():
          start_feat(b, cchunk + 2)

    wait_tagw(0, NCHUNK - 2)
    wait_tagw(1, NCHUNK - 1)

    @pl.when(half == 0)
    def _():
      wait_catw(0, NCHUNK - 2)
      wait_catw(1, NCHUNK - 1)

  return seq_emb


_SEQ_EMB = _build_kernel()


def kernel(item_seq, feat_category, feat_tags, cat_table, tag_table):
  n_items = feat_category.shape[0]
  feat_all = jnp.concatenate(
      [feat_tags, feat_category[:, None],
       jnp.zeros((n_items, FEAT_W - TAG_LEN - 1), _i32)], axis=1)
  tt_pairs = jax.lax.bitcast_convert_type(
      tag_table.astype(jnp.bfloat16).reshape(TAG_V, DH, 2), _i32)
  ttab_t = tt_pairs.T.reshape(2, DH // 2 * TAG_V)
  out_cat, out_tag = _SEQ_EMB(item_seq.reshape(T), feat_all, cat_table,
                              ttab_t)
  return out_cat.reshape(B, L, D), out_tag.reshape(B, L, D)
